# baseline (speedup 1.0000x reference)
import jax
import jax.numpy as jnp
from jax import lax
from jax.experimental import pallas as pl
from jax.experimental.pallas import tpu as pltpu
from jax.experimental.pallas import tpu_sc as plsc

B, C, H, W = 16, 384, 64, 64
D = H * W
R = B * C
NC, NS = 2, 16
NW = NC * NS
RPW = R // NW
CPW = C // NC
L = 16
NG = RPW // L


def _body(x_hbm, perm_hbm, out_hbm, perm_v, sem):
    wid = lax.axis_index("s") * NC + lax.axis_index("c")
    b = wid // 2
    c0 = (wid % 2) * CPW
    base = wid * RPW

    pltpu.sync_copy(perm_hbm, perm_v)

    def group(i, carry):
        src = perm_v[pl.ds(c0 + L * i, L)] + b * C
        for l in range(L):
            s = src[l]
            pltpu.make_async_copy(
                x_hbm.at[pl.ds(s, 1)],
                out_hbm.at[pl.ds(base + L * i + l, 1)],
                sem,
            ).start()
        return carry

    lax.fori_loop(0, NG, group, 0)

    def drain(i, carry):
        pltpu.make_async_copy(
            x_hbm.at[pl.ds(0, 1)], out_hbm.at[pl.ds(base, 1)], sem
        ).wait()
        return carry

    lax.fori_loop(0, RPW, drain, 0)


@jax.jit
def _shuffle(x2d, perm32):
    mesh = plsc.VectorSubcoreMesh(
        core_axis_name="c", subcore_axis_name="s", num_cores=NC, num_subcores=NS
    )
    f = pl.kernel(
        _body,
        out_type=jax.ShapeDtypeStruct((R, D), jnp.float32),
        mesh=mesh,
        scratch_types=[
            pltpu.VMEM((C,), jnp.int32),
            pltpu.SemaphoreType.DMA,
        ],
    )
    return f(x2d, perm32)


def kernel(inputs, permutation):
    x2d = inputs.reshape(R, D)
    perm32 = permutation.astype(jnp.int32)
    return _shuffle(x2d, perm32).reshape(B, C, H, W)


# speedup vs baseline: 6.1937x; 6.1937x over previous
import jax
import jax.numpy as jnp
from jax import lax
from jax.experimental import pallas as pl
from jax.experimental.pallas import tpu as pltpu
from jax.experimental.pallas import tpu_sc as plsc

B, C, H, W = 16, 384, 64, 64
D = H * W
R = B * C
NC, NS = 2, 16
NW = NC * NS
RPW = R // NW
CPW = C // NC
L = 16
CHUNK = 8
RING = 3
NCHUNK = RPW // CHUNK
CPB = 6
NBODY = NCHUNK // CPB
SROWS = RING * CHUNK


def _body(x_hbm, perm_hbm, out_hbm, perm_v, spm, g0, g1, g2, s0, s1, s2):
    wid = lax.axis_index("s") * NC + lax.axis_index("c")
    sid = lax.axis_index("s")
    b = wid // 2
    c0 = (wid % 2) * CPW
    base = wid * RPW
    sbase = sid * SROWS

    gsems = (g0, g1, g2)
    ssems = (s0, s1, s2)

    pltpu.sync_copy(perm_hbm, perm_v)

    def gwait(slot):
        pltpu.make_async_copy(
            x_hbm.at[pl.ds(0, CHUNK)],
            spm.at[pl.ds(sbase + slot * CHUNK, CHUNK)],
            gsems[slot],
        ).wait()

    def scatter(k, slot):
        return pltpu.make_async_copy(
            spm.at[pl.ds(sbase + slot * CHUNK, CHUNK)],
            out_hbm.at[pl.ds(base + k * CHUNK, CHUNK)],
            ssems[slot],
        )

    def body(o, carry):
        k0 = 6 * o
        srcv = None
        for j in range(CPB):
            k = k0 + j
            slot = j % 3
            if j % 2 == 0:
                g = 3 * o + j // 2
                srcv = perm_v[pl.ds(c0 + L * g, L)] + b * C

            if j >= 3:
                scatter(k - 3, slot).wait()
            else:

                @pl.when(o > 0)
                def _():
                    scatter(k - 3, slot).wait()

            for l in range(CHUNK):
                s = srcv[(j % 2) * CHUNK + l]
                pltpu.make_async_copy(
                    x_hbm.at[pl.ds(s, 1)],
                    spm.at[pl.ds(sbase + slot * CHUNK + l, 1)],
                    gsems[slot],
                ).start()

            pslot = (j + 2) % 3
            if j >= 1:
                gwait(pslot)
                scatter(k - 1, pslot).start()
            else:

                @pl.when(o > 0)
                def _():
                    gwait(pslot)
                    scatter(k - 1, pslot).start()

        return carry

    lax.fori_loop(0, NBODY, body, 0)

    gwait((NCHUNK - 1) % 3)
    scatter(NCHUNK - 1, (NCHUNK - 1) % 3).start()
    scatter(NCHUNK - 3, (NCHUNK - 3) % 3).wait()
    scatter(NCHUNK - 2, (NCHUNK - 2) % 3).wait()
    scatter(NCHUNK - 1, (NCHUNK - 1) % 3).wait()


@jax.jit
def _shuffle(x2d, perm32):
    mesh = plsc.VectorSubcoreMesh(
        core_axis_name="c", subcore_axis_name="s", num_cores=NC, num_subcores=NS
    )
    f = pl.kernel(
        _body,
        out_type=jax.ShapeDtypeStruct((R, D), jnp.float32),
        mesh=mesh,
        scratch_types=[
            pltpu.VMEM((C,), jnp.int32),
            pltpu.VMEM_SHARED((NS * SROWS, D), jnp.float32),
            pltpu.SemaphoreType.DMA,
            pltpu.SemaphoreType.DMA,
            pltpu.SemaphoreType.DMA,
            pltpu.SemaphoreType.DMA,
            pltpu.SemaphoreType.DMA,
            pltpu.SemaphoreType.DMA,
        ],
    )
    return f(x2d, perm32)


def kernel(inputs, permutation):
    x2d = inputs.reshape(R, D)
    perm32 = permutation.astype(jnp.int32)
    return _shuffle(x2d, perm32).reshape(B, C, H, W)
